# baseline (speedup 1.0000x reference)
import jax
import jax.numpy as jnp
from jax.experimental import pallas as pl
from jax.experimental.pallas import tpu as pltpu

N = 10000
D_IN = 128
D_OUT = 32

ROW_BLOCK = 200
NB = N // ROW_BLOCK


def _fused_kernel(x_ref, wt_ref, adj_ref, ot_ref, xw_ref, tile_ref):
    i = pl.program_id(0)

    @pl.when(i == 0)
    def _():
        xw_ref[...] = jax.lax.dot_general(
            x_ref[...], wt_ref[...],
            dimension_numbers=(((1,), (1,)), ((), ())),
            preferred_element_type=jnp.float32,
        ).astype(jnp.bfloat16)

    def store_t(c):
        ot_ref[:, c * ROW_BLOCK:(c + 1) * ROW_BLOCK] = tile_ref[...].T

    for c in range(NB - 1):
        pl.when(i == c + 1)(lambda c=c: store_t(c))

    acc = jax.lax.dot_general(
        adj_ref[...], xw_ref[...],
        dimension_numbers=(((1,), (0,)), ((), ())),
        preferred_element_type=jnp.float32,
    )
    tile_ref[...] = jnp.maximum(acc, 0.0)

    @pl.when(i == NB - 1)
    def _():
        store_t(NB - 1)


def kernel(inputs, adj_norm, weight):
    grid = (NB,)
    out_t = pl.pallas_call(
        _fused_kernel,
        grid=grid,
        in_specs=[
            pl.BlockSpec((N, D_IN), lambda i: (0, 0)),
            pl.BlockSpec((D_OUT, D_IN), lambda i: (0, 0)),
            pl.BlockSpec((ROW_BLOCK, N), lambda i: (i, 0)),
        ],
        out_specs=pl.BlockSpec((D_OUT, N), lambda i: (0, 0)),
        out_shape=jax.ShapeDtypeStruct((D_OUT, N), jnp.float32),
        scratch_shapes=[
            pltpu.VMEM((N, D_OUT), jnp.bfloat16),
            pltpu.VMEM((ROW_BLOCK, D_OUT), jnp.float32),
        ],
        compiler_params=pltpu.CompilerParams(
            dimension_semantics=("arbitrary",),
        ),
    )(inputs, weight.T, adj_norm)
    return out_t.T


# speedup vs baseline: 1.0404x; 1.0404x over previous
import jax
import jax.numpy as jnp
from jax.experimental import pallas as pl
from jax.experimental.pallas import tpu as pltpu

N = 10000
D_IN = 128
D_OUT = 32

ROW_BLOCK = 400
NB = N // ROW_BLOCK


def _adj_copy(adj_hbm, buf_ref, sem, block, slot):
    return pltpu.make_async_copy(
        adj_hbm.at[pl.ds(block * ROW_BLOCK, ROW_BLOCK), :],
        buf_ref.at[slot],
        sem.at[slot],
    )


def _fused_kernel(x_hbm, wt_ref, adj_hbm, ot_ref,
                  xw_ref, tile_ref, x_ref, buf_ref, adj_sem, x_sem):
    i = pl.program_id(0)
    slot = jax.lax.rem(i, 2)

    @pl.when(i == 0)
    def _():
        pltpu.make_async_copy(x_hbm, x_ref, x_sem).start()
        _adj_copy(adj_hbm, buf_ref, adj_sem, 0, 0).start()
        _adj_copy(adj_hbm, buf_ref, adj_sem, 1, 1).start()
        pltpu.make_async_copy(x_hbm, x_ref, x_sem).wait()
        xw_ref[...] = jax.lax.dot_general(
            x_ref[...], wt_ref[...],
            dimension_numbers=(((1,), (1,)), ((), ())),
            preferred_element_type=jnp.float32,
        ).astype(jnp.bfloat16)

    def store_t(c):
        ot_ref[:, c * ROW_BLOCK:(c + 1) * ROW_BLOCK] = tile_ref[...].T

    for c in range(NB - 1):
        pl.when(i == c + 1)(lambda c=c: store_t(c))

    _adj_copy(adj_hbm, buf_ref, adj_sem, i, slot).wait()
    acc = jax.lax.dot_general(
        buf_ref[slot], xw_ref[...],
        dimension_numbers=(((1,), (0,)), ((), ())),
        preferred_element_type=jnp.float32,
    )
    tile_ref[...] = jnp.maximum(acc, 0.0)

    @pl.when(i < NB - 2)
    def _():
        _adj_copy(adj_hbm, buf_ref, adj_sem, i + 2, slot).start()

    @pl.when(i == NB - 1)
    def _():
        store_t(NB - 1)


def kernel(inputs, adj_norm, weight):
    grid = (NB,)
    out_t = pl.pallas_call(
        _fused_kernel,
        grid=grid,
        in_specs=[
            pl.BlockSpec(memory_space=pl.ANY),
            pl.BlockSpec((D_OUT, D_IN), lambda i: (0, 0)),
            pl.BlockSpec(memory_space=pl.ANY),
        ],
        out_specs=pl.BlockSpec((D_OUT, N), lambda i: (0, 0)),
        out_shape=jax.ShapeDtypeStruct((D_OUT, N), jnp.float32),
        scratch_shapes=[
            pltpu.VMEM((N, D_OUT), jnp.bfloat16),
            pltpu.VMEM((ROW_BLOCK, D_OUT), jnp.float32),
            pltpu.VMEM((N, D_IN), jnp.float32),
            pltpu.VMEM((2, ROW_BLOCK, N), jnp.float32),
            pltpu.SemaphoreType.DMA((2,)),
            pltpu.SemaphoreType.DMA,
        ],
        compiler_params=pltpu.CompilerParams(
            dimension_semantics=("arbitrary",),
        ),
    )(inputs, weight.T, adj_norm)
    return out_t.T


# speedup vs baseline: 1.0447x; 1.0042x over previous
import jax
import jax.numpy as jnp
from jax.experimental import pallas as pl
from jax.experimental.pallas import tpu as pltpu

N = 10000
D_IN = 128
D_OUT = 32

ROW_BLOCK = 400
NB = N // ROW_BLOCK


def _fused_kernel(x_ref, wt_ref, adj_ref, ot_ref, xw_ref, tile_ref):
    i = pl.program_id(0)

    @pl.when(i == 0)
    def _():
        xw_ref[...] = jax.lax.dot_general(
            x_ref[...], wt_ref[...],
            dimension_numbers=(((1,), (1,)), ((), ())),
            preferred_element_type=jnp.float32,
        ).astype(jnp.bfloat16)

    def store_t(c):
        ot_ref[:, c * ROW_BLOCK:(c + 1) * ROW_BLOCK] = tile_ref[...].T

    for c in range(NB - 1):
        pl.when(i == c + 1)(lambda c=c: store_t(c))

    acc = jax.lax.dot_general(
        adj_ref[...], xw_ref[...],
        dimension_numbers=(((1,), (0,)), ((), ())),
        preferred_element_type=jnp.float32,
    )
    tile_ref[...] = jnp.maximum(acc, 0.0)

    @pl.when(i == NB - 1)
    def _():
        store_t(NB - 1)


def kernel(inputs, adj_norm, weight):
    grid = (NB,)
    out_t = pl.pallas_call(
        _fused_kernel,
        grid=grid,
        in_specs=[
            pl.BlockSpec((N, D_IN), lambda i: (0, 0)),
            pl.BlockSpec((D_OUT, D_IN), lambda i: (0, 0)),
            pl.BlockSpec((ROW_BLOCK, N), lambda i: (i, 0)),
        ],
        out_specs=pl.BlockSpec((D_OUT, N), lambda i: (0, 0)),
        out_shape=jax.ShapeDtypeStruct((D_OUT, N), jnp.float32),
        scratch_shapes=[
            pltpu.VMEM((N, D_OUT), jnp.bfloat16),
            pltpu.VMEM((ROW_BLOCK, D_OUT), jnp.float32),
        ],
        compiler_params=pltpu.CompilerParams(
            dimension_semantics=("arbitrary",),
        ),
    )(inputs, weight.T, adj_norm)
    return out_t.T
